# Initial kernel scaffold; baseline (speedup 1.0000x reference)
import jax
import jax.numpy as jnp
from jax.experimental import pallas as pl


def kernel(x, edge_index, W1, b1, W2, b2, Wf_pre, bf_pre, Wf_upd, bf_upd, Wb_pre, bb_pre, Wb_upd, bb_upd):
    raise NotImplementedError("write your pallas kernel here")


# speedup vs baseline: 6.2901x; 6.2901x over previous
import jax
import jax.numpy as jnp
from jax import lax
from jax.experimental import pallas as pl
from jax.experimental.pallas import tpu as pltpu
from jax.experimental.pallas import tpu_sc as plsc

N = 10000
E = 320000
D = 128
H = 64
W = 128
NB = E // 128
NC = 2
NS = 16
NPAD = 10240
ROWS_PER_TILE = NPAD // NS


def _pre_body(x_ref, w1_ref, b1_ref, w2_ref, b2_ref,
              wf_ref, bf_ref, wb_ref, bb_ref,
              y_ref, mf_ref, mb_ref):
    x = x_ref[...]
    h = jnp.maximum(jnp.dot(x, w1_ref[...],
                            preferred_element_type=jnp.float32) + b1_ref[...], 0.0)
    y = jnp.dot(h, w2_ref[...], preferred_element_type=jnp.float32) + b2_ref[...]
    y_ref[...] = y
    mf = jnp.maximum(jnp.dot(y, wf_ref[...],
                             preferred_element_type=jnp.float32) + bf_ref[...], 0.0)
    mb = jnp.maximum(jnp.dot(y, wb_ref[...],
                             preferred_element_type=jnp.float32) + bb_ref[...], 0.0)
    rows = mf.shape[0]
    pad = jnp.concatenate(
        [jnp.ones((rows, 1), jnp.float32), jnp.zeros((rows, W - H - 1), jnp.float32)],
        axis=1)
    mf_ref[...] = jnp.concatenate([mf, pad], axis=1)
    mb_ref[...] = jnp.concatenate([mb, pad], axis=1)


def _dense_pre(x, W1, b1, W2, b2, Wf_pre, bf_pre, Wb_pre, bb_pre):
    blk = 1000
    grid = N // blk
    full = lambda shape: pl.BlockSpec(shape, lambda i: (0,) * len(shape))
    return pl.pallas_call(
        _pre_body,
        grid=(grid,),
        in_specs=[
            pl.BlockSpec((blk, D), lambda i: (i, 0)),
            full((D, D)), full((D,)), full((D, H)), full((H,)),
            full((H, H)), full((H,)), full((H, H)), full((H,)),
        ],
        out_specs=[
            pl.BlockSpec((blk, H), lambda i: (i, 0)),
            pl.BlockSpec((blk, W), lambda i: (i, 0)),
            pl.BlockSpec((blk, W), lambda i: (i, 0)),
        ],
        out_shape=[
            jax.ShapeDtypeStruct((N, H), jnp.float32),
            jax.ShapeDtypeStruct((N, W), jnp.float32),
            jax.ShapeDtypeStruct((N, W), jnp.float32),
        ],
    )(x, W1, b1, W2, b2, Wf_pre, bf_pre, Wb_pre, bb_pre)


def _sc_body(mf_hbm, mb_hbm, src_hbm, dst_hbm, zero_hbm,
             zf_out, zb_out,
             gidx, sidx, rows_v,
             z_sh,
             sem_g, sem_s):
    c = lax.axis_index("c")
    s = lax.axis_index("s")

    row0 = s * ROWS_PER_TILE
    pltpu.sync_copy(zero_hbm, z_sh.at[pl.ds(row0, ROWS_PER_TILE)])
    plsc.subcore_barrier()

    per = NB // NS
    rem = NB - per * NS
    start = s * per + jnp.minimum(s, rem)
    nb = per + jnp.where(s < rem, 1, 0)

    @pl.when(c == 0)
    def _():
        def body(i, carry):
            b = start + i
            pltpu.sync_copy(src_hbm.at[b], gidx)
            pltpu.sync_copy(dst_hbm.at[b], sidx)
            pltpu.async_copy(mf_hbm.at[gidx], rows_v, sem_g).wait()
            pltpu.async_copy(rows_v, z_sh.at[sidx], sem_s, add=True).wait()
            return carry
        lax.fori_loop(0, nb, body, 0)

    @pl.when(c == 1)
    def _():
        def body(i, carry):
            b = start + i
            pltpu.sync_copy(dst_hbm.at[b], gidx)
            pltpu.sync_copy(src_hbm.at[b], sidx)
            pltpu.async_copy(mb_hbm.at[gidx], rows_v, sem_g).wait()
            pltpu.async_copy(rows_v, z_sh.at[sidx], sem_s, add=True).wait()
            return carry
        lax.fori_loop(0, nb, body, 0)

    plsc.subcore_barrier()

    @pl.when(c == 0)
    def _():
        pltpu.sync_copy(z_sh.at[pl.ds(row0, ROWS_PER_TILE)],
                        zf_out.at[pl.ds(row0, ROWS_PER_TILE)])

    @pl.when(c == 1)
    def _():
        pltpu.sync_copy(z_sh.at[pl.ds(row0, ROWS_PER_TILE)],
                        zb_out.at[pl.ds(row0, ROWS_PER_TILE)])


def _sc_scatter(mf, mb, src2, dst2, zero):
    mesh = plsc.VectorSubcoreMesh(core_axis_name="c", subcore_axis_name="s")
    kern = pl.kernel(
        _sc_body,
        out_type=(
            jax.ShapeDtypeStruct((NPAD, W), jnp.float32),
            jax.ShapeDtypeStruct((NPAD, W), jnp.float32),
        ),
        mesh=mesh,
        scratch_types=[
            pltpu.VMEM((128,), jnp.int32),
            pltpu.VMEM((128,), jnp.int32),
            pltpu.VMEM((128, W), jnp.float32),
            pltpu.VMEM_SHARED((NPAD, W), jnp.float32),
            pltpu.SemaphoreType.DMA,
            pltpu.SemaphoreType.DMA,
        ],
    )
    return kern(mf, mb, src2, dst2, zero)


def _post_body(y_ref, zf_ref, zb_ref, wf_ref, bf_ref, wb_ref, bb_ref, out_ref):
    y = y_ref[...]
    af = zf_ref[...]
    ab = zb_ref[...]
    zf = af[:, :H] / jnp.maximum(af[:, H:H + 1], 1.0)
    zb = ab[:, :H] / jnp.maximum(ab[:, H:H + 1], 1.0)
    hf = jnp.maximum(jnp.dot(zf, wf_ref[...],
                             preferred_element_type=jnp.float32) + bf_ref[...], 0.0) + y
    hb = jnp.maximum(jnp.dot(zb, wb_ref[...],
                             preferred_element_type=jnp.float32) + bb_ref[...], 0.0) + y
    out_ref[...] = jnp.concatenate([hf, hb], axis=1)


def _dense_post(y, zf, zb, Wf_upd, bf_upd, Wb_upd, bb_upd):
    blk = 1000
    grid = N // blk
    full = lambda shape: pl.BlockSpec(shape, lambda i: (0,) * len(shape))
    return pl.pallas_call(
        _post_body,
        grid=(grid,),
        in_specs=[
            pl.BlockSpec((blk, H), lambda i: (i, 0)),
            pl.BlockSpec((blk, W), lambda i: (i, 0)),
            pl.BlockSpec((blk, W), lambda i: (i, 0)),
            full((H, H)), full((H,)), full((H, H)), full((H,)),
        ],
        out_specs=pl.BlockSpec((blk, 2 * H), lambda i: (i, 0)),
        out_shape=jax.ShapeDtypeStruct((N, 2 * H), jnp.float32),
    )(y, zf, zb, Wf_upd, bf_upd, Wb_upd, bb_upd)


@jax.jit
def kernel(x, edge_index, W1, b1, W2, b2,
           Wf_pre, bf_pre, Wf_upd, bf_upd,
           Wb_pre, bb_pre, Wb_upd, bb_upd):
    y, mf, mb = _dense_pre(x, W1, b1, W2, b2, Wf_pre, bf_pre, Wb_pre, bb_pre)
    src2 = edge_index[0].reshape(NB, 128)
    dst2 = edge_index[1].reshape(NB, 128)
    zero = jnp.zeros((ROWS_PER_TILE, W), jnp.float32)
    zf, zb = _sc_scatter(mf, mb, src2, dst2, zero)
    return _dense_post(y, zf, zb, Wf_upd, bf_upd, Wb_upd, bb_upd)
